# baseline (speedup 1.0000x reference)
import functools

import numpy as np
import jax
import jax.numpy as jnp
from jax import lax
from jax.experimental import pallas as pl
from jax.experimental.pallas import tpu as pltpu
from jax.experimental.pallas import tpu_sc as plsc

D_MODEL = 128
MAX_LEN = 200
BATCH = 4096
N = BATCH * MAX_LEN
NUM_CORES = 2
NUM_SUBCORES = 16
NW = NUM_CORES * NUM_SUBCORES
RPW = N // NW
CH = MAX_LEN
NCH = RPW // CH
NBUF = 4


def _pe_table():
    even = np.arange(0, D_MODEL, 2, dtype=np.float32)
    inv = np.reciprocal(
        np.power(np.float32(10000.0), even / np.float32(D_MODEL))
    ).astype(np.float32)
    pos = np.arange(MAX_LEN, dtype=np.float32).reshape(MAX_LEN, 1)
    ang = (pos * inv.reshape(1, D_MODEL // 2)).astype(np.float32)
    pe = np.empty((MAX_LEN, D_MODEL), dtype=np.float32)
    pe[:, 0::2] = np.sin(ang)
    pe[:, 1::2] = np.cos(ang)
    return jnp.asarray(pe)


_mesh = plsc.VectorSubcoreMesh(core_axis_name="c", subcore_axis_name="s")


@functools.partial(
    pl.kernel,
    mesh=_mesh,
    out_type=jax.ShapeDtypeStruct((N, D_MODEL), jnp.float32),
    scratch_types=[
        pltpu.VMEM((CH,), jnp.int32),
        pltpu.VMEM((CH,), jnp.int32),
        pltpu.VMEM((CH,), jnp.int32),
        pltpu.VMEM((CH,), jnp.int32),
        pltpu.VMEM((NBUF, CH, D_MODEL), jnp.float32),
        pltpu.VMEM((MAX_LEN, D_MODEL), jnp.float32),
        pltpu.SemaphoreType.DMA,
        pltpu.SemaphoreType.DMA,
        pltpu.SemaphoreType.DMA,
        pltpu.SemaphoreType.DMA,
        pltpu.SemaphoreType.DMA,
        pltpu.SemaphoreType.DMA,
        pltpu.SemaphoreType.DMA,
        pltpu.SemaphoreType.DMA,
        pltpu.SemaphoreType.DMA,
        pltpu.SemaphoreType.DMA,
        pltpu.SemaphoreType.DMA,
        pltpu.SemaphoreType.DMA,
    ],
)
def _emb_kernel(tokens_hbm, table_hbm, pe_hbm, out_hbm,
                i0, i1, i2, i3, rows_v, pe_v,
                t0, t1, t2, t3, g0, g1, g2, g3, s0, s1, s2, s3):
    idxs = (i0, i1, i2, i3)
    wid = lax.axis_index("s") * NUM_CORES + lax.axis_index("c")
    wbase = wid * RPW
    pltpu.sync_copy(pe_hbm, pe_v)

    tsems = (t0, t1, t2, t3)
    gsems = (g0, g1, g2, g3)
    ssems = (s0, s1, s2, s3)

    def token(c, b):
        return pltpu.make_async_copy(
            tokens_hbm.at[pl.ds(wbase + c * CH, CH)], idxs[b], tsems[b])

    def gather(c, b):
        return pltpu.make_async_copy(
            table_hbm.at[idxs[b]], rows_v.at[b], gsems[b])

    def store(c, b):
        return pltpu.make_async_copy(
            rows_v.at[b], out_hbm.at[pl.ds(wbase + c * CH, CH)], ssems[b])

    for c in range(NBUF):
        token(c, c).start()
    for c in range(2):
        token(c, c).wait()
        gather(c, c).start()

    def body(c4, carry):
        for b in range(NBUF):
            c = c4 * NBUF + b
            bn = (b + 2) % NBUF

            @pl.when(c >= 2)
            def _():
                store(c - 2, bn).wait()

            @pl.when(c + 2 < NCH)
            def _():
                token(c + 2, bn).wait()
                gather(c + 2, bn).start()

            gather(c, b).wait()

            @pl.when(c + NBUF < NCH)
            def _():
                token(c + NBUF, b).start()

            def add_pe(r4, rcarry):
                for u in range(4):
                    r = r4 * 4 + u
                    for g in range(D_MODEL // 16):
                        s = pl.ds(g * 16, 16)
                        plsc.addupdate(rows_v.at[b, r, s], pe_v[r, s])
                return rcarry

            lax.fori_loop(0, CH // 4, add_pe, 0)
            store(c, b).start()
        return carry

    lax.fori_loop(0, NCH // NBUF, body, 0)
    store(NCH - 2, (NCH - 2) % NBUF).wait()
    store(NCH - 1, (NCH - 1) % NBUF).wait()


def kernel(tokens, emb_table):
    pe = _pe_table()
    out = _emb_kernel(tokens.reshape(N), emb_table, pe)
    return out.reshape(BATCH, MAX_LEN, D_MODEL)


# speedup vs baseline: 1.0114x; 1.0114x over previous
import functools

import numpy as np
import jax
import jax.numpy as jnp
from jax import lax
from jax.experimental import pallas as pl
from jax.experimental.pallas import tpu as pltpu
from jax.experimental.pallas import tpu_sc as plsc

D_MODEL = 128
MAX_LEN = 200
BATCH = 4096
N = BATCH * MAX_LEN
NUM_CORES = 2
NUM_SUBCORES = 16
NW = NUM_CORES * NUM_SUBCORES
RPW = N // NW
CH = MAX_LEN
NCH = RPW // CH
NBUF = 3
MAIN = (NCH // NBUF) * NBUF


def _pe_table():
    even = np.arange(0, D_MODEL, 2, dtype=np.float32)
    inv = np.reciprocal(
        np.power(np.float32(10000.0), even / np.float32(D_MODEL))
    ).astype(np.float32)
    pos = np.arange(MAX_LEN, dtype=np.float32).reshape(MAX_LEN, 1)
    ang = (pos * inv.reshape(1, D_MODEL // 2)).astype(np.float32)
    pe = np.empty((MAX_LEN, D_MODEL), dtype=np.float32)
    pe[:, 0::2] = np.sin(ang)
    pe[:, 1::2] = np.cos(ang)
    return jnp.asarray(pe)


_mesh = plsc.VectorSubcoreMesh(core_axis_name="c", subcore_axis_name="s")


@functools.partial(
    pl.kernel,
    mesh=_mesh,
    out_type=jax.ShapeDtypeStruct((N, D_MODEL), jnp.float32),
    scratch_types=[
        pltpu.VMEM((RPW,), jnp.int32),
        pltpu.VMEM((NBUF, CH, D_MODEL), jnp.float32),
        pltpu.VMEM((MAX_LEN, D_MODEL), jnp.float32),
        pltpu.SemaphoreType.DMA,
        pltpu.SemaphoreType.DMA,
        pltpu.SemaphoreType.DMA,
        pltpu.SemaphoreType.DMA,
        pltpu.SemaphoreType.DMA,
        pltpu.SemaphoreType.DMA,
        pltpu.SemaphoreType.DMA,
    ],
)
def _emb_kernel(tokens_hbm, table_hbm, pe_hbm, out_hbm, idx_v, rows_v, pe_v,
                psem, g0, g1, g2, s0, s1, s2):
    wid = lax.axis_index("s") * NUM_CORES + lax.axis_index("c")
    wbase = wid * RPW

    pe_copy = pltpu.make_async_copy(pe_hbm, pe_v, psem)
    pe_copy.start()
    pltpu.sync_copy(tokens_hbm.at[pl.ds(wbase, RPW)], idx_v)

    gsems = (g0, g1, g2)
    ssems = (s0, s1, s2)

    def gather(c, b):
        return pltpu.make_async_copy(
            table_hbm.at[idx_v.at[pl.ds(c * CH, CH)]], rows_v.at[b], gsems[b])

    def store(c, b):
        return pltpu.make_async_copy(
            rows_v.at[b], out_hbm.at[pl.ds(wbase + c * CH, CH)], ssems[b])

    def step(c, b, last):
        bn = (b + 1) % NBUF

        @pl.when(c >= 2)
        def _():
            store(c - 2, bn).wait()

        if last:
            @pl.when(c + 1 < NCH)
            def _():
                gather(c + 1, bn).start()
        else:
            gather(c + 1, bn).start()

        gather(c, b).wait()

        def add_pe(r4, rcarry):
            for u in range(4):
                r = r4 * 4 + u
                for g in range(D_MODEL // 16):
                    s = pl.ds(g * 16, 16)
                    plsc.addupdate(rows_v.at[b, r, s], pe_v[r, s])
            return rcarry

        lax.fori_loop(0, CH // 4, add_pe, 0)
        store(c, b).start()

    gather(0, 0).start()
    pe_copy.wait()

    def body(c3, carry):
        for b in range(NBUF):
            step(c3 * NBUF + b, b, last=False)
        return carry

    lax.fori_loop(0, MAIN // NBUF, body, 0)
    for c in range(MAIN, NCH):
        step(c, c % NBUF, last=(c == NCH - 1))
    store(NCH - 2, (NCH - 2) % NBUF).wait()
    store(NCH - 1, (NCH - 1) % NBUF).wait()


def kernel(tokens, emb_table):
    pe = _pe_table()
    out = _emb_kernel(tokens.reshape(N), emb_table, pe)
    return out.reshape(BATCH, MAX_LEN, D_MODEL)
